# baseline (speedup 1.0000x reference)
import jax
import jax.numpy as jnp
from jax.experimental import pallas as pl
from jax.experimental.pallas import tpu as pltpu

_NC = 21
_BG = 20
_RATIO = 3
_N = 131072
_BL = 16384
_NB = _N // _BL


def _cumsum_lanes(x, size):
    d = 1
    while d < size:
        pad = jnp.zeros(x.shape[:-1] + (d,), x.dtype)
        x = x + jnp.concatenate([pad, x[..., :-d]], axis=-1)
        d *= 2
    return x


def _fold_lanes(x, size):
    n = size
    while n > 128:
        h = n // 2
        x = x[..., :h] + x[..., h:n]
        n = h
    return x


def _ssd_kernel(gt_ref, cats_ref, bbs_ref, gtb_ref,
                tot_ref, loc_ref, conf_ref, iacc, cacc, lacc):
    i = pl.program_id(0)

    @pl.when(i == 0)
    def _init():
        iacc[0] = jnp.sum((gt_ref[...] != _BG).astype(jnp.int32))
        iacc[1] = 0
        cacc[...] = jnp.zeros_like(cacc)
        lacc[...] = jnp.zeros_like(lacc)

    gts = gt_ref[pl.ds(i * _BL, _BL)]
    neg = gts == _BG
    pos = jnp.logical_not(neg)
    posf = pos.astype(jnp.float32)
    negi = neg.astype(jnp.int32)
    inc = _cumsum_lanes(negi, _BL)
    rank = inc - negi + iacc[1]
    k = iacc[0] * _RATIO
    sel = jnp.logical_and(neg, rank < k)
    w = jnp.logical_or(pos, sel).astype(jnp.float32)
    iacc[1] = iacc[1] + jnp.sum(negi)

    xt = cats_ref[...]
    cls = jax.lax.broadcasted_iota(jnp.int32, (_NC, _BL), 0)
    t = jnp.logical_and(cls == gts, pos)
    sp = jnp.maximum(xt, 0.0) + jnp.log1p(jnp.exp(-jnp.abs(xt)))
    contrib = jnp.where(cls < (_NC - 1),
                        sp * w - jnp.where(t, xt, 0.0), 0.0)
    cacc[...] = cacc[...] + _fold_lanes(contrib, _BL)

    d = bbs_ref[...] - gtb_ref[...]
    ad = jnp.abs(d)
    sl1 = jnp.where(ad < 1.0, 0.5 * d * d, ad - 0.5)
    lacc[...] = lacc[...] + _fold_lanes(sl1 * posf, _BL)

    @pl.when(i == _NB - 1)
    def _fin():
        n = iacc[0].astype(jnp.float32)
        conf = jnp.sum(cacc[...])
        loc = jnp.sum(lacc[...])
        loc_ref[0, 0] = loc
        conf_ref[0, 0] = conf
        tot_ref[0, 0] = (1.0 / n) * (conf + loc)


def kernel(bbs_preds, cats_preds, gt_bbs, gt_cats):
    gt1 = gt_cats.astype(jnp.int32)
    tot, loc, conf = pl.pallas_call(
        _ssd_kernel,
        grid=(_NB,),
        in_specs=[
            pl.BlockSpec((_N,), lambda i: (0,)),
            pl.BlockSpec((_NC, _BL), lambda i: (0, i)),
            pl.BlockSpec((4, _BL), lambda i: (0, i)),
            pl.BlockSpec((4, _BL), lambda i: (0, i)),
        ],
        out_specs=[pl.BlockSpec(memory_space=pltpu.SMEM)] * 3,
        out_shape=[jax.ShapeDtypeStruct((1, 1), jnp.float32)] * 3,
        scratch_shapes=[pltpu.SMEM((2,), jnp.int32),
                        pltpu.VMEM((_NC, 128), jnp.float32),
                        pltpu.VMEM((4, 128), jnp.float32)],
        compiler_params=pltpu.CompilerParams(
            dimension_semantics=("arbitrary",)),
    )(gt1, cats_preds.T, bbs_preds.T, gt_bbs.T)
    return (tot[0, 0], loc[0, 0], conf[0, 0])
